# baseline (speedup 1.0000x reference)
import functools

import jax
import jax.numpy as jnp
from jax import lax
from jax.experimental import pallas as pl
from jax.experimental.pallas import tpu as pltpu
from jax.experimental.pallas import tpu_sc as plsc

B = 16384
D = 128
C = 100000
NC = 2
NS = 16
NW = NC * NS
L = 16
RPW = B // NW
CH = 128
NCHUNK = RPW // CH
NBUF = 3
HPW = 6256
HIST = NS * HPW
LPH = B // NS
LROWS = LPH // 128


def _loss_body(feature_hbm, label2d_hbm, center_hbm, out_hbm,
               lblh, lblr, fbuf, cbuf, nbuf, obuf, ones, zbuf, hist,
               semz, seml, semlr, sema, semo,
               semn0, semn1, semn2, semn3,
               semf0, semf1, semf2, semc0, semc1, semc2):
    semf = [semf0, semf1, semf2]
    semc = [semc0, semc1, semc2]
    semn = [semn0, semn1, semn2, semn3]
    c = lax.axis_index("c")
    s = lax.axis_index("s")
    wid = s * NC + c
    base = wid * RPW

    dlr = pltpu.async_copy(label2d_hbm.at[pl.ds(wid * NCHUNK, NCHUNK)],
                           lblr, semlr)
    dlh = pltpu.async_copy(label2d_hbm.at[pl.ds(s * LROWS, LROWS)],
                           lblh, seml)
    df = [None] * NCHUNK
    dc = [None] * NCHUNK
    H = CH // 2

    def _fire_f(k, slot):
        return [pltpu.async_copy(
            feature_hbm.at[pl.ds(base + k * CH + h * H, H)],
            fbuf.at[slot, pl.ds(h * H, H)], semf[slot]) for h in range(2)]

    def _fire_c(k, slot):
        return [pltpu.async_copy(
            center_hbm.at[lblr.at[k, pl.ds(h * H, H)]],
            cbuf.at[slot, pl.ds(h * H, H)], semc[slot]) for h in range(2)]

    df[0] = _fire_f(0, 0)
    dlr.wait()
    dc[0] = _fire_c(0, 0)
    for k in range(1, min(NBUF, NCHUNK)):
        df[k] = _fire_f(k, k)
        dc[k] = _fire_c(k, k)

    zero16 = jnp.zeros((L,), jnp.float32)
    one16 = jnp.ones((L,), jnp.float32)
    for g in range(128 // L):
        ones[pl.ds(g * L, L)] = one16

    def _zfill(i, _):
        zbuf[pl.ds(i * L, L)] = zero16
        return 0
    lax.fori_loop(0, HPW // L, _zfill, 0)

    pltpu.async_copy(zbuf, hist.at[pl.ds(s * HPW, HPW)], semz).wait()
    dlh.wait()
    plsc.subcore_barrier()
    da = [pltpu.async_copy(ones, hist.at[lblh.at[j]], sema, add=True)
          for j in range(LROWS)]
    for d in da:
        d.wait()
    plsc.subcore_barrier()

    dn = [pltpu.async_copy(hist.at[lblr.at[k]], nbuf.at[k], semn[k])
          for k in range(NCHUNK)]

    iota = lax.iota(jnp.int32, L)
    rowids = [iota + (g * L) for g in range(CH // L)]

    do = [None] * NCHUNK
    for k in range(NCHUNK):
        slot = k % NBUF
        for d in dc[k]:
            d.wait()
        for d in df[k]:
            d.wait()
        fb = fbuf.at[slot]
        cb = cbuf.at[slot]

        def _col(j, accs, fb=fb, cb=cb):
            jv = jnp.full((L,), j, jnp.int32)
            colv = (jv & ~(L - 1)) + ((iota + jv) & (L - 1))
            new = []
            for g in range(CH // L):
                gf = plsc.load_gather(fb, [rowids[g], colv])
                gc = plsc.load_gather(cb, [rowids[g], colv])
                dv = gf - gc
                new.append(accs[g] + dv * dv)
            return tuple(new)

        accs = lax.fori_loop(0, D, _col,
                             tuple(zero16 for _ in range(CH // L)))

        dn[k].wait()
        for g in range(CH // L):
            d2 = accs[g]
            n = nbuf[k, pl.ds(g * L, L)]
            yi = jnp.int32(0x5F3759DF) - lax.shift_right_logical(
                plsc.bitcast(d2, jnp.int32), 1)
            y = plsc.bitcast(yi, jnp.float32)
            for _ in range(3):
                y = y * (1.5 - 0.5 * d2 * y * y)
            dist = jnp.where(d2 > 0.0, d2 * y, 0.0)
            obuf[k, pl.ds(g * L, L)] = dist / n

        do[k] = pltpu.async_copy(obuf.at[k], out_hbm.at[pl.ds(base + k * CH, CH)],
                                 semo)

        nxt = k + NBUF
        if nxt < NCHUNK:
            df[nxt] = _fire_f(nxt, slot)
            dc[nxt] = _fire_c(nxt, slot)

    for k in range(NCHUNK):
        do[k].wait()


@jax.jit
def _center_loss(feature, label, center):
    mesh = plsc.VectorSubcoreMesh(core_axis_name="c", subcore_axis_name="s",
                                  num_cores=NC, num_subcores=NS)
    run = functools.partial(
        pl.kernel,
        out_type=jax.ShapeDtypeStruct((B,), jnp.float32),
        mesh=mesh,
        compiler_params=pltpu.CompilerParams(needs_layout_passes=False),
        scratch_types=[
            pltpu.VMEM((LROWS, 128), jnp.int32),
            pltpu.VMEM((NCHUNK, 128), jnp.int32),
            pltpu.VMEM((NBUF, CH, D), jnp.float32),
            pltpu.VMEM((NBUF, CH, D), jnp.float32),
            pltpu.VMEM((NCHUNK, 128), jnp.float32),
            pltpu.VMEM((NCHUNK, CH), jnp.float32),
            pltpu.VMEM((128,), jnp.float32),
            pltpu.VMEM((HPW,), jnp.float32),
            pltpu.VMEM_SHARED((HIST,), jnp.float32),
        ] + [pltpu.SemaphoreType.DMA] * 15,
    )(_loss_body)
    label2d = label.astype(jnp.int32).reshape(B // 128, 128)
    return run(feature, label2d, center)


def kernel(feature, label, center):
    return _center_loss(feature, label, center)


# speedup vs baseline: 1.0129x; 1.0129x over previous
import functools

import jax
import jax.numpy as jnp
from jax import lax
from jax.experimental import pallas as pl
from jax.experimental.pallas import tpu as pltpu
from jax.experimental.pallas import tpu_sc as plsc

B = 16384
D = 128
C = 100000
NC = 2
NS = 16
NW = NC * NS
L = 16
RPW = B // NW
CH = 128
NCHUNK = RPW // CH
NBUF = 3
HPW = 6256
HIST = NS * HPW
LPH = B // NS
LROWS = LPH // 128


def _loss_body(feature_hbm, label2d_hbm, center_hbm, out_hbm,
               lblh, lblr, fbuf, cbuf, nbuf, obuf, ones, zbuf, hist,
               semz, seml, semlr, sema, semo,
               semn0, semn1, semn2, semn3,
               semf0, semf1, semf2, semc0, semc1, semc2):
    semf = [semf0, semf1, semf2]
    semc = [semc0, semc1, semc2]
    semn = [semn0, semn1, semn2, semn3]
    c = lax.axis_index("c")
    s = lax.axis_index("s")
    wid = s * NC + c
    base = wid * RPW

    dlr = pltpu.async_copy(label2d_hbm.at[pl.ds(wid * NCHUNK, NCHUNK)],
                           lblr, semlr)
    dlh = pltpu.async_copy(label2d_hbm.at[pl.ds(s * LROWS, LROWS)],
                           lblh, seml)
    df = [None] * NCHUNK
    dc = [None] * NCHUNK
    df[0] = pltpu.async_copy(feature_hbm.at[pl.ds(base, CH)],
                             fbuf.at[0], semf[0])
    dlr.wait()
    dc[0] = pltpu.async_copy(center_hbm.at[lblr.at[0]], cbuf.at[0], semc[0])
    for k in range(1, min(NBUF, NCHUNK)):
        df[k] = pltpu.async_copy(
            feature_hbm.at[pl.ds(base + k * CH, CH)], fbuf.at[k], semf[k])
        dc[k] = pltpu.async_copy(
            center_hbm.at[lblr.at[k]], cbuf.at[k], semc[k])

    zero16 = jnp.zeros((L,), jnp.float32)
    one16 = jnp.ones((L,), jnp.float32)
    for g in range(128 // L):
        ones[pl.ds(g * L, L)] = one16

    def _zfill(i, _):
        zbuf[pl.ds(i * L, L)] = zero16
        return 0
    lax.fori_loop(0, HPW // L, _zfill, 0)

    pltpu.async_copy(zbuf, hist.at[pl.ds(s * HPW, HPW)], semz).wait()
    dlh.wait()
    plsc.subcore_barrier()
    da = [pltpu.async_copy(ones, hist.at[lblh.at[j]], sema, add=True)
          for j in range(LROWS)]
    for d in da:
        d.wait()
    plsc.subcore_barrier()

    dn = [pltpu.async_copy(hist.at[lblr.at[k]], nbuf.at[k], semn[k])
          for k in range(NCHUNK)]

    iota = lax.iota(jnp.int32, L)
    rowids = [iota + (g * L) for g in range(CH // L)]

    do = [None] * NCHUNK
    for k in range(NCHUNK):
        slot = k % NBUF
        dc[k].wait()
        df[k].wait()
        fb = fbuf.at[slot]
        cb = cbuf.at[slot]

        def _col(j, accs, fb=fb, cb=cb):
            jv = jnp.full((L,), j, jnp.int32)
            colv = (jv & ~(L - 1)) + ((iota + jv) & (L - 1))
            new = []
            for g in range(CH // L):
                gf = plsc.load_gather(fb, [rowids[g], colv])
                gc = plsc.load_gather(cb, [rowids[g], colv])
                dv = gf - gc
                new.append(accs[g] + dv * dv)
            return tuple(new)

        accs = lax.fori_loop(0, D, _col,
                             tuple(zero16 for _ in range(CH // L)))

        dn[k].wait()
        for g in range(CH // L):
            d2 = accs[g]
            n = nbuf[k, pl.ds(g * L, L)]
            yi = jnp.int32(0x5F3759DF) - lax.shift_right_logical(
                plsc.bitcast(d2, jnp.int32), 1)
            y = plsc.bitcast(yi, jnp.float32)
            for _ in range(3):
                y = y * (1.5 - 0.5 * d2 * y * y)
            dist = jnp.where(d2 > 0.0, d2 * y, 0.0)
            obuf[k, pl.ds(g * L, L)] = dist / n

        do[k] = pltpu.async_copy(obuf.at[k], out_hbm.at[pl.ds(base + k * CH, CH)],
                                 semo)

        nxt = k + NBUF
        if nxt < NCHUNK:
            df[nxt] = pltpu.async_copy(
                feature_hbm.at[pl.ds(base + nxt * CH, CH)],
                fbuf.at[slot], semf[slot])
            dc[nxt] = pltpu.async_copy(
                center_hbm.at[lblr.at[nxt]], cbuf.at[slot], semc[slot])

    for k in range(NCHUNK):
        do[k].wait()


@jax.jit
def _center_loss(feature, label, center):
    mesh = plsc.VectorSubcoreMesh(core_axis_name="c", subcore_axis_name="s",
                                  num_cores=NC, num_subcores=NS)
    run = functools.partial(
        pl.kernel,
        out_type=jax.ShapeDtypeStruct((B,), jnp.float32),
        mesh=mesh,
        compiler_params=pltpu.CompilerParams(needs_layout_passes=False),
        scratch_types=[
            pltpu.VMEM((LROWS, 128), jnp.int32),
            pltpu.VMEM((NCHUNK, 128), jnp.int32),
            pltpu.VMEM((NBUF, CH, D), jnp.float32),
            pltpu.VMEM((NBUF, CH, D), jnp.float32),
            pltpu.VMEM((NCHUNK, 128), jnp.float32),
            pltpu.VMEM((NCHUNK, CH), jnp.float32),
            pltpu.VMEM((128,), jnp.float32),
            pltpu.VMEM((HPW,), jnp.float32),
            pltpu.VMEM_SHARED((HIST,), jnp.float32),
        ] + [pltpu.SemaphoreType.DMA] * 15,
    )(_loss_body)
    label2d = label.astype(jnp.int32).reshape(B // 128, 128)
    return run(feature, label2d, center)


def kernel(feature, label, center):
    return _center_loss(feature, label, center)
